# baseline (speedup 1.0000x reference)
import functools

import jax
import jax.numpy as jnp
from jax import lax
from jax.experimental import pallas as pl
from jax.experimental.pallas import tpu as pltpu
from jax.experimental.pallas import tpu_sc as plsc

E = 16
DM = 1024
DF = 2048
T = 2048
TK = 2
P = T * TK
BT = 256
NB = 32
NPAD = NB * BT
NFF = 2
FT = DF // NFF

RR, RC = 32, 128
_SC_CHUNK = 64
_NCH = P // _SC_CHUNK


def _route_body(e_ref, dst_ref, tok_ref, meta_ref):
    e = e_ref[...]
    dst = jnp.zeros((RR, RC), jnp.int32)
    pe = 0
    pe_list = []
    for ex in range(E):
        mi = (e == ex).astype(jnp.int32)
        cs = mi
        k = 1
        while k < RC:
            cs = cs + jnp.concatenate(
                [jnp.zeros((RR, k), jnp.int32), cs[:, : RC - k]], axis=1)
            k *= 2
        row_tot = cs[:, RC - 1 : RC]
        rt = row_tot
        k = 1
        while k < RR:
            rt = rt + jnp.concatenate(
                [jnp.zeros((k, 1), jnp.int32), rt[: RR - k, :]], axis=0)
            k *= 2
        excl = (cs - mi) + (rt - row_tot)
        cnt = jnp.sum(mi)
        pc = ((cnt + BT - 1) // BT) * BT
        dst = dst + mi * (excl + pe)
        pe = pe + pc
        pe_list.append(pe)
    nv = pe // BT
    dst_ref[...] = dst

    i2 = (lax.broadcasted_iota(jnp.int32, (RR, RC), 0) * RC
          + lax.broadcasted_iota(jnp.int32, (RR, RC), 1))
    tok_ref[...] = i2 // TK

    raw = []
    for b in range(NB):
        rb = 0
        for ex in range(E):
            rb = rb + (pe_list[ex] <= b * BT).astype(jnp.int32)
        raw.append(rb)
    be_last = 0
    for b in range(NB):
        be_last = be_last + jnp.where(b == nv - 1, raw[b], 0)
    lane = lax.broadcasted_iota(jnp.int32, (8, 64), 1)
    meta = jnp.zeros((8, 64), jnp.int32)
    for b in range(NB):
        val = jnp.where(b < nv, raw[b], be_last)
        meta = meta + val * (lane == b).astype(jnp.int32)
    meta = meta + nv * (lane == NB).astype(jnp.int32)
    meta_ref[...] = meta


def _route(e2d):
    return pl.pallas_call(
        _route_body,
        in_specs=[pl.BlockSpec((RR, RC), lambda: (0, 0))],
        out_specs=[
            pl.BlockSpec((RR, RC), lambda: (0, 0)),
            pl.BlockSpec((RR, RC), lambda: (0, 0)),
            pl.BlockSpec((8, 64), lambda: (0, 0)),
        ],
        out_shape=[
            jax.ShapeDtypeStruct((RR, RC), jnp.int32),
            jax.ShapeDtypeStruct((RR, RC), jnp.int32),
            jax.ShapeDtypeStruct((8, 64), jnp.int32),
        ],
    )(e2d)


def _sc_dispatch_rows(table, ptok2, dst2):
    info = plsc.get_sparse_core_info()
    nw = info.num_cores * info.num_subcores
    ch_per_w = _NCH // nw
    mesh = plsc.VectorSubcoreMesh(core_axis_name="c", subcore_axis_name="s")

    @functools.partial(
        pl.kernel,
        mesh=mesh,
        out_type=jax.ShapeDtypeStruct((NPAD, DM), jnp.float32),
        scratch_types=[
            pltpu.VMEM((_SC_CHUNK,), jnp.int32),
            pltpu.VMEM((_SC_CHUNK,), jnp.int32),
            pltpu.VMEM((_SC_CHUNK, DM), jnp.float32),
            pltpu.SemaphoreType.DMA,
        ],
    )
    def k(table_hbm, ti_hbm, di_hbm, out_hbm, ti_v, di_v, rows_v, sem):
        wid = lax.axis_index("s") * info.num_cores + lax.axis_index("c")

        @pl.loop(0, ch_per_w)
        def _(j):
            row = wid * ch_per_w + j
            pltpu.sync_copy(ti_hbm.at[row], ti_v)
            pltpu.sync_copy(di_hbm.at[row], di_v)
            pltpu.async_copy(table_hbm.at[ti_v], rows_v, sem).wait()
            pltpu.async_copy(rows_v, out_hbm.at[di_v], sem).wait()

    return k(table, ptok2, dst2)


def _sc_gather_rows(table, idx, n_rows, d):
    info = plsc.get_sparse_core_info()
    nw = info.num_cores * info.num_subcores
    r_per_w = n_rows // nw
    mesh = plsc.VectorSubcoreMesh(core_axis_name="c", subcore_axis_name="s")

    @functools.partial(
        pl.kernel,
        mesh=mesh,
        out_type=jax.ShapeDtypeStruct((n_rows, d), jnp.float32),
        scratch_types=[
            pltpu.VMEM((r_per_w,), jnp.int32),
            pltpu.VMEM((_SC_CHUNK, d), jnp.float32),
            pltpu.SemaphoreType.DMA,
        ],
    )
    def k(table_hbm, idx_hbm, out_hbm, idx_v, rows_v, sem):
        wid = lax.axis_index("s") * info.num_cores + lax.axis_index("c")
        base = wid * r_per_w
        pltpu.sync_copy(idx_hbm.at[pl.ds(base, r_per_w)], idx_v)

        @pl.loop(0, r_per_w, step=_SC_CHUNK)
        def _(j):
            pltpu.async_copy(
                table_hbm.at[idx_v.at[pl.ds(j, _SC_CHUNK)]], rows_v, sem
            ).wait()
            pltpu.sync_copy(rows_v, out_hbm.at[pl.ds(base + j, _SC_CHUNK)])

    return k(table, idx)


def _ffn_body(meta_ref, x_ref, wg_ref, wu_ref, wd_ref, y_ref, acc_ref):
    b = pl.program_id(0)
    f = pl.program_id(1)
    nv = meta_ref[0, NB]

    @pl.when(b < nv)
    def _():
        x = x_ref[...].astype(jnp.bfloat16)
        gate = jnp.dot(x, wg_ref[0].astype(jnp.bfloat16),
                       preferred_element_type=jnp.float32)
        up = jnp.dot(x, wu_ref[0].astype(jnp.bfloat16),
                     preferred_element_type=jnp.float32)
        h = (gate * jax.nn.sigmoid(gate) * up).astype(jnp.bfloat16)
        part = jnp.dot(h, wd_ref[0].astype(jnp.bfloat16),
                       preferred_element_type=jnp.float32)

        @pl.when(f == 0)
        def _():
            acc_ref[...] = part

        @pl.when(f != 0)
        def _():
            acc_ref[...] += part

        @pl.when(f == NFF - 1)
        def _():
            y_ref[...] = acc_ref[...]


def _grouped_ffn(meta, xg, Wg, Wu, Wd):
    def _ff(b, f, m):
        nv = m[0, NB]
        serp = jnp.where(b % 2 == 0, f, NFF - 1 - f)
        return jnp.where(b < nv, serp, nv % 2)

    def _blk(b, m):
        return jnp.minimum(b, m[0, NB] - 1)

    grid_spec = pltpu.PrefetchScalarGridSpec(
        num_scalar_prefetch=1,
        grid=(NB, NFF),
        in_specs=[
            pl.BlockSpec((BT, DM), lambda b, f, m: (_blk(b, m), 0)),
            pl.BlockSpec((1, DM, FT), lambda b, f, m: (m[0, b], 0, _ff(b, f, m))),
            pl.BlockSpec((1, DM, FT), lambda b, f, m: (m[0, b], 0, _ff(b, f, m))),
            pl.BlockSpec((1, FT, DM), lambda b, f, m: (m[0, b], _ff(b, f, m), 0)),
        ],
        out_specs=pl.BlockSpec(
            (BT, DM), lambda b, f, m: (jnp.where(b < m[0, NB], b, NB - 1), 0)
        ),
        scratch_shapes=[pltpu.VMEM((BT, DM), jnp.float32)],
    )
    return pl.pallas_call(
        _ffn_body,
        grid_spec=grid_spec,
        out_shape=jax.ShapeDtypeStruct((NPAD, DM), jnp.float32),
    )(meta, xg, Wg, Wu, Wd)


def _pair_sum_body(g_ref, w_ref, o_ref):
    g = g_ref[...]
    w = w_ref[...]
    o_ref[...] = g[:, :DM] * w[:, 0:1] + g[:, DM:] * w[:, 1:2]


def _pair_sum(g2, w):
    return pl.pallas_call(
        _pair_sum_body,
        grid=(T // BT,),
        in_specs=[
            pl.BlockSpec((BT, TK * DM), lambda i: (i, 0)),
            pl.BlockSpec((BT, TK), lambda i: (i, 0)),
        ],
        out_specs=pl.BlockSpec((BT, DM), lambda i: (i, 0)),
        out_shape=jax.ShapeDtypeStruct((T, DM), jnp.float32),
    )(g2, w)


def kernel(hidden_states, top_k_index, top_k_weights, Wg, Wu, Wd):
    e2d = top_k_index.astype(jnp.int32).reshape(RR, RC)
    dst2d, tok2d, meta = _route(e2d)

    xg = _sc_dispatch_rows(
        hidden_states,
        tok2d.reshape(_NCH, _SC_CHUNK),
        dst2d.reshape(_NCH, _SC_CHUNK),
    )
    y = _grouped_ffn(meta, xg, Wg, Wu, Wd)
    g = _sc_gather_rows(y, dst2d.reshape(P), P, DM)
    return _pair_sum(g.reshape(T, TK * DM),
                     top_k_weights.astype(jnp.float32))


# speedup vs baseline: 1.0447x; 1.0447x over previous
import functools

import jax
import jax.numpy as jnp
from jax import lax
from jax.experimental import pallas as pl
from jax.experimental.pallas import tpu as pltpu
from jax.experimental.pallas import tpu_sc as plsc

E = 16
DM = 1024
DF = 2048
T = 2048
TK = 2
P = T * TK
BT = 256
NB = 32
NPAD = NB * BT
NFF = 1
FT = DF // NFF

RR, RC = 32, 128
_SC_CHUNK = 64
_NCH = P // _SC_CHUNK


def _route_body(e_ref, dst_ref, tok_ref, meta_ref):
    e = e_ref[...]
    dst = jnp.zeros((RR, RC), jnp.int32)
    pe = 0
    pe_list = []
    for ex in range(E):
        mi = (e == ex).astype(jnp.int32)
        cs = mi
        k = 1
        while k < RC:
            cs = cs + jnp.concatenate(
                [jnp.zeros((RR, k), jnp.int32), cs[:, : RC - k]], axis=1)
            k *= 2
        row_tot = cs[:, RC - 1 : RC]
        rt = row_tot
        k = 1
        while k < RR:
            rt = rt + jnp.concatenate(
                [jnp.zeros((k, 1), jnp.int32), rt[: RR - k, :]], axis=0)
            k *= 2
        excl = (cs - mi) + (rt - row_tot)
        cnt = jnp.sum(mi)
        pc = ((cnt + BT - 1) // BT) * BT
        dst = dst + mi * (excl + pe)
        pe = pe + pc
        pe_list.append(pe)
    nv = pe // BT
    dst_ref[...] = dst

    i2 = (lax.broadcasted_iota(jnp.int32, (RR, RC), 0) * RC
          + lax.broadcasted_iota(jnp.int32, (RR, RC), 1))
    tok_ref[...] = i2 // TK

    raw = []
    for b in range(NB):
        rb = 0
        for ex in range(E):
            rb = rb + (pe_list[ex] <= b * BT).astype(jnp.int32)
        raw.append(rb)
    be_last = 0
    for b in range(NB):
        be_last = be_last + jnp.where(b == nv - 1, raw[b], 0)
    lane = lax.broadcasted_iota(jnp.int32, (8, 64), 1)
    meta = jnp.zeros((8, 64), jnp.int32)
    for b in range(NB):
        val = jnp.where(b < nv, raw[b], be_last)
        meta = meta + val * (lane == b).astype(jnp.int32)
    meta = meta + nv * (lane == NB).astype(jnp.int32)
    meta_ref[...] = meta


def _route(e2d):
    return pl.pallas_call(
        _route_body,
        in_specs=[pl.BlockSpec((RR, RC), lambda: (0, 0))],
        out_specs=[
            pl.BlockSpec((RR, RC), lambda: (0, 0)),
            pl.BlockSpec((RR, RC), lambda: (0, 0)),
            pl.BlockSpec((8, 64), lambda: (0, 0)),
        ],
        out_shape=[
            jax.ShapeDtypeStruct((RR, RC), jnp.int32),
            jax.ShapeDtypeStruct((RR, RC), jnp.int32),
            jax.ShapeDtypeStruct((8, 64), jnp.int32),
        ],
    )(e2d)


def _sc_dispatch_rows(table, ptok2, dst2):
    info = plsc.get_sparse_core_info()
    nw = info.num_cores * info.num_subcores
    ch_per_w = _NCH // nw
    mesh = plsc.VectorSubcoreMesh(core_axis_name="c", subcore_axis_name="s")

    @functools.partial(
        pl.kernel,
        mesh=mesh,
        out_type=jax.ShapeDtypeStruct((NPAD, DM), jnp.float32),
        scratch_types=[
            pltpu.VMEM((_SC_CHUNK,), jnp.int32),
            pltpu.VMEM((_SC_CHUNK,), jnp.int32),
            pltpu.VMEM((_SC_CHUNK, DM), jnp.float32),
            pltpu.SemaphoreType.DMA,
        ],
    )
    def k(table_hbm, ti_hbm, di_hbm, out_hbm, ti_v, di_v, rows_v, sem):
        wid = lax.axis_index("s") * info.num_cores + lax.axis_index("c")

        @pl.loop(0, ch_per_w)
        def _(j):
            row = wid * ch_per_w + j
            pltpu.sync_copy(ti_hbm.at[row], ti_v)
            pltpu.sync_copy(di_hbm.at[row], di_v)
            pltpu.async_copy(table_hbm.at[ti_v], rows_v, sem).wait()
            pltpu.async_copy(rows_v, out_hbm.at[di_v], sem).wait()

    return k(table, ptok2, dst2)


def _sc_gather_rows(table, idx, n_rows, d):
    info = plsc.get_sparse_core_info()
    nw = info.num_cores * info.num_subcores
    r_per_w = n_rows // nw
    mesh = plsc.VectorSubcoreMesh(core_axis_name="c", subcore_axis_name="s")

    @functools.partial(
        pl.kernel,
        mesh=mesh,
        out_type=jax.ShapeDtypeStruct((n_rows, d), jnp.float32),
        scratch_types=[
            pltpu.VMEM((r_per_w,), jnp.int32),
            pltpu.VMEM((_SC_CHUNK, d), jnp.float32),
            pltpu.SemaphoreType.DMA,
        ],
    )
    def k(table_hbm, idx_hbm, out_hbm, idx_v, rows_v, sem):
        wid = lax.axis_index("s") * info.num_cores + lax.axis_index("c")
        base = wid * r_per_w
        pltpu.sync_copy(idx_hbm.at[pl.ds(base, r_per_w)], idx_v)

        @pl.loop(0, r_per_w, step=_SC_CHUNK)
        def _(j):
            pltpu.async_copy(
                table_hbm.at[idx_v.at[pl.ds(j, _SC_CHUNK)]], rows_v, sem
            ).wait()
            pltpu.sync_copy(rows_v, out_hbm.at[pl.ds(base + j, _SC_CHUNK)])

    return k(table, idx)


def _ffn_body(meta_ref, x_ref, wg_ref, wu_ref, wd_ref, y_ref, acc_ref):
    b = pl.program_id(0)
    f = pl.program_id(1)
    nv = meta_ref[0, NB]

    @pl.when(b < nv)
    def _():
        x = x_ref[...].astype(jnp.bfloat16)
        gate = jnp.dot(x, wg_ref[0].astype(jnp.bfloat16),
                       preferred_element_type=jnp.float32)
        up = jnp.dot(x, wu_ref[0].astype(jnp.bfloat16),
                     preferred_element_type=jnp.float32)
        h = (gate * jax.nn.sigmoid(gate) * up).astype(jnp.bfloat16)
        part = jnp.dot(h, wd_ref[0].astype(jnp.bfloat16),
                       preferred_element_type=jnp.float32)

        @pl.when(f == 0)
        def _():
            acc_ref[...] = part

        @pl.when(f != 0)
        def _():
            acc_ref[...] += part

        @pl.when(f == NFF - 1)
        def _():
            y_ref[...] = acc_ref[...]


def _grouped_ffn(meta, xg, Wg, Wu, Wd):
    def _ff(b, f, m):
        nv = m[0, NB]
        serp = jnp.where(b % 2 == 0, f, NFF - 1 - f)
        return jnp.where(b < nv, serp, (nv % 2) * (NFF - 1))

    def _blk(b, m):
        return jnp.minimum(b, m[0, NB] - 1)

    grid_spec = pltpu.PrefetchScalarGridSpec(
        num_scalar_prefetch=1,
        grid=(NB, NFF),
        in_specs=[
            pl.BlockSpec((BT, DM), lambda b, f, m: (_blk(b, m), 0)),
            pl.BlockSpec((1, DM, FT), lambda b, f, m: (m[0, b], 0, _ff(b, f, m))),
            pl.BlockSpec((1, DM, FT), lambda b, f, m: (m[0, b], 0, _ff(b, f, m))),
            pl.BlockSpec((1, FT, DM), lambda b, f, m: (m[0, b], _ff(b, f, m), 0)),
        ],
        out_specs=pl.BlockSpec(
            (BT, DM), lambda b, f, m: (jnp.where(b < m[0, NB], b, NB - 1), 0)
        ),
        scratch_shapes=[pltpu.VMEM((BT, DM), jnp.float32)],
    )
    return pl.pallas_call(
        _ffn_body,
        grid_spec=grid_spec,
        out_shape=jax.ShapeDtypeStruct((NPAD, DM), jnp.float32),
    )(meta, xg, Wg, Wu, Wd)


def _pair_sum_body(g_ref, w_ref, o_ref):
    g = g_ref[...]
    w = w_ref[...]
    o_ref[...] = g[:, :DM] * w[:, 0:1] + g[:, DM:] * w[:, 1:2]


def _pair_sum(g2, w):
    return pl.pallas_call(
        _pair_sum_body,
        grid=(T // BT,),
        in_specs=[
            pl.BlockSpec((BT, TK * DM), lambda i: (i, 0)),
            pl.BlockSpec((BT, TK), lambda i: (i, 0)),
        ],
        out_specs=pl.BlockSpec((BT, DM), lambda i: (i, 0)),
        out_shape=jax.ShapeDtypeStruct((T, DM), jnp.float32),
    )(g2, w)


def kernel(hidden_states, top_k_index, top_k_weights, Wg, Wu, Wd):
    e2d = top_k_index.astype(jnp.int32).reshape(RR, RC)
    dst2d, tok2d, meta = _route(e2d)

    xg = _sc_dispatch_rows(
        hidden_states,
        tok2d.reshape(_NCH, _SC_CHUNK),
        dst2d.reshape(_NCH, _SC_CHUNK),
    )
    y = _grouped_ffn(meta, xg, Wg, Wu, Wd)
    g = _sc_gather_rows(y, dst2d.reshape(P), P, DM)
    return _pair_sum(g.reshape(T, TK * DM),
                     top_k_weights.astype(jnp.float32))


# speedup vs baseline: 1.0580x; 1.0128x over previous
import functools

import jax
import jax.numpy as jnp
from jax import lax
from jax.experimental import pallas as pl
from jax.experimental.pallas import tpu as pltpu
from jax.experimental.pallas import tpu_sc as plsc

E = 16
DM = 1024
DF = 2048
T = 2048
TK = 2
P = T * TK
BT = 256
NB = 32
NPAD = NB * BT
NFF = 1
FT = DF // NFF

RR, RC = 32, 128
_SC_CHUNK = 64
_NCH = P // _SC_CHUNK


def _route_body(e_ref, dst_ref, tok_ref, meta_ref):
    e = e_ref[...]
    dst = jnp.zeros((RR, RC), jnp.int32)
    pe = 0
    pe_list = []
    for ex in range(E):
        mi = (e == ex).astype(jnp.int32)
        cs = mi
        k = 1
        while k < RC:
            cs = cs + jnp.concatenate(
                [jnp.zeros((RR, k), jnp.int32), cs[:, : RC - k]], axis=1)
            k *= 2
        row_tot = cs[:, RC - 1 : RC]
        rt = row_tot
        k = 1
        while k < RR:
            rt = rt + jnp.concatenate(
                [jnp.zeros((k, 1), jnp.int32), rt[: RR - k, :]], axis=0)
            k *= 2
        excl = (cs - mi) + (rt - row_tot)
        cnt = jnp.sum(mi)
        pc = ((cnt + BT - 1) // BT) * BT
        dst = dst + mi * (excl + pe)
        pe = pe + pc
        pe_list.append(pe)
    nv = pe // BT
    dst_ref[...] = dst

    i2 = (lax.broadcasted_iota(jnp.int32, (RR, RC), 0) * RC
          + lax.broadcasted_iota(jnp.int32, (RR, RC), 1))
    tok_ref[...] = i2 // TK

    raw = []
    for b in range(NB):
        rb = 0
        for ex in range(E):
            rb = rb + (pe_list[ex] <= b * BT).astype(jnp.int32)
        raw.append(rb)
    be_last = 0
    for b in range(NB):
        be_last = be_last + jnp.where(b == nv - 1, raw[b], 0)
    lane = lax.broadcasted_iota(jnp.int32, (8, 64), 1)
    meta = jnp.zeros((8, 64), jnp.int32)
    for b in range(NB):
        val = jnp.where(b < nv, raw[b], be_last)
        meta = meta + val * (lane == b).astype(jnp.int32)
    meta = meta + nv * (lane == NB).astype(jnp.int32)
    meta_ref[...] = meta


def _route(e2d):
    return pl.pallas_call(
        _route_body,
        in_specs=[pl.BlockSpec((RR, RC), lambda: (0, 0))],
        out_specs=[
            pl.BlockSpec((RR, RC), lambda: (0, 0)),
            pl.BlockSpec((RR, RC), lambda: (0, 0)),
            pl.BlockSpec((8, 64), lambda: (0, 0)),
        ],
        out_shape=[
            jax.ShapeDtypeStruct((RR, RC), jnp.int32),
            jax.ShapeDtypeStruct((RR, RC), jnp.int32),
            jax.ShapeDtypeStruct((8, 64), jnp.int32),
        ],
    )(e2d)


def _sc_dispatch_rows(table, ptok2, dst2):
    info = plsc.get_sparse_core_info()
    nw = info.num_cores * info.num_subcores
    ch_per_w = _NCH // nw
    mesh = plsc.VectorSubcoreMesh(core_axis_name="c", subcore_axis_name="s")

    @functools.partial(
        pl.kernel,
        mesh=mesh,
        out_type=jax.ShapeDtypeStruct((NPAD, DM), jnp.float32),
        scratch_types=[
            pltpu.VMEM((_SC_CHUNK,), jnp.int32),
            pltpu.VMEM((_SC_CHUNK,), jnp.int32),
            pltpu.VMEM((_SC_CHUNK, DM), jnp.float32),
            pltpu.SemaphoreType.DMA,
        ],
    )
    def k(table_hbm, ti_hbm, di_hbm, out_hbm, ti_v, di_v, rows_v, sem):
        wid = lax.axis_index("s") * info.num_cores + lax.axis_index("c")

        @pl.loop(0, ch_per_w)
        def _(j):
            row = wid * ch_per_w + j
            pltpu.sync_copy(ti_hbm.at[row], ti_v)
            pltpu.sync_copy(di_hbm.at[row], di_v)
            pltpu.async_copy(table_hbm.at[ti_v], rows_v, sem).wait()
            pltpu.async_copy(rows_v, out_hbm.at[di_v], sem).wait()

    return k(table, ptok2, dst2)


def _sc_gather_rows(table, idx, n_rows, d):
    info = plsc.get_sparse_core_info()
    nw = info.num_cores * info.num_subcores
    r_per_w = n_rows // nw
    mesh = plsc.VectorSubcoreMesh(core_axis_name="c", subcore_axis_name="s")

    @functools.partial(
        pl.kernel,
        mesh=mesh,
        out_type=jax.ShapeDtypeStruct((n_rows, d), jnp.float32),
        scratch_types=[
            pltpu.VMEM((r_per_w,), jnp.int32),
            pltpu.VMEM((_SC_CHUNK, d), jnp.float32),
            pltpu.SemaphoreType.DMA,
        ],
    )
    def k(table_hbm, idx_hbm, out_hbm, idx_v, rows_v, sem):
        wid = lax.axis_index("s") * info.num_cores + lax.axis_index("c")
        base = wid * r_per_w
        pltpu.sync_copy(idx_hbm.at[pl.ds(base, r_per_w)], idx_v)

        @pl.loop(0, r_per_w, step=_SC_CHUNK)
        def _(j):
            pltpu.async_copy(
                table_hbm.at[idx_v.at[pl.ds(j, _SC_CHUNK)]], rows_v, sem
            ).wait()
            pltpu.sync_copy(rows_v, out_hbm.at[pl.ds(base + j, _SC_CHUNK)])

    return k(table, idx)


def _ffn_body(meta_ref, x_ref, wg_ref, wu_ref, wd_ref, y_ref, acc_ref):
    b = pl.program_id(0)
    f = pl.program_id(1)
    nv = meta_ref[0, NB]

    @pl.when(b < nv)
    def _():
        x = x_ref[...].astype(jnp.bfloat16)
        gate = jnp.dot(x, wg_ref[0].astype(jnp.bfloat16),
                       preferred_element_type=jnp.float32)
        up = jnp.dot(x, wu_ref[0].astype(jnp.bfloat16),
                     preferred_element_type=jnp.float32)
        h = (gate * jax.nn.sigmoid(gate) * up).astype(jnp.bfloat16)
        part = jnp.dot(h, wd_ref[0].astype(jnp.bfloat16),
                       preferred_element_type=jnp.float32)

        if NFF == 1:
            y_ref[...] = part
        else:
            @pl.when(f == 0)
            def _():
                acc_ref[...] = part

            @pl.when(f != 0)
            def _():
                acc_ref[...] += part

            @pl.when(f == NFF - 1)
            def _():
                y_ref[...] = acc_ref[...]


def _grouped_ffn(meta, xg, Wg, Wu, Wd):
    def _ff(b, f, m):
        nv = m[0, NB]
        serp = jnp.where(b % 2 == 0, f, NFF - 1 - f)
        return jnp.where(b < nv, serp, (nv % 2) * (NFF - 1))

    def _blk(b, m):
        return jnp.minimum(b, m[0, NB] - 1)

    grid_spec = pltpu.PrefetchScalarGridSpec(
        num_scalar_prefetch=1,
        grid=(NB, NFF),
        in_specs=[
            pl.BlockSpec((BT, DM), lambda b, f, m: (_blk(b, m), 0)),
            pl.BlockSpec((1, DM, FT), lambda b, f, m: (m[0, b], 0, _ff(b, f, m))),
            pl.BlockSpec((1, DM, FT), lambda b, f, m: (m[0, b], 0, _ff(b, f, m))),
            pl.BlockSpec((1, FT, DM), lambda b, f, m: (m[0, b], _ff(b, f, m), 0)),
        ],
        out_specs=pl.BlockSpec(
            (BT, DM), lambda b, f, m: (jnp.where(b < m[0, NB], b, NB - 1), 0)
        ),
        scratch_shapes=[pltpu.VMEM((BT, DM), jnp.float32)],
    )
    return pl.pallas_call(
        _ffn_body,
        grid_spec=grid_spec,
        out_shape=jax.ShapeDtypeStruct((NPAD, DM), jnp.float32),
    )(meta, xg, Wg, Wu, Wd)


def _pair_sum_body(g_ref, w_ref, o_ref):
    g = g_ref[...]
    w = w_ref[...]
    o_ref[...] = g[:, :DM] * w[:, 0:1] + g[:, DM:] * w[:, 1:2]


def _pair_sum(g2, w):
    return pl.pallas_call(
        _pair_sum_body,
        grid=(T // BT,),
        in_specs=[
            pl.BlockSpec((BT, TK * DM), lambda i: (i, 0)),
            pl.BlockSpec((BT, TK), lambda i: (i, 0)),
        ],
        out_specs=pl.BlockSpec((BT, DM), lambda i: (i, 0)),
        out_shape=jax.ShapeDtypeStruct((T, DM), jnp.float32),
    )(g2, w)


def kernel(hidden_states, top_k_index, top_k_weights, Wg, Wu, Wd):
    e2d = top_k_index.astype(jnp.int32).reshape(RR, RC)
    dst2d, tok2d, meta = _route(e2d)

    xg = _sc_dispatch_rows(
        hidden_states,
        tok2d.reshape(_NCH, _SC_CHUNK),
        dst2d.reshape(_NCH, _SC_CHUNK),
    )
    y = _grouped_ffn(meta, xg, Wg, Wu, Wd)
    g = _sc_gather_rows(y, dst2d.reshape(P), P, DM)
    return _pair_sum(g.reshape(T, TK * DM),
                     top_k_weights.astype(jnp.float32))
